# baseline (speedup 1.0000x reference)
import jax
import jax.numpy as jnp
from jax.experimental import pallas as pl

_HOP = 256
_SEG = 512
_KT = 512


def _frames_kernel(x_ref, xh_ref, w_ref, o_ref):
    v3 = x_ref[...].reshape(8, _KT, _HOP)
    vh = xh_ref[...].reshape(8, 1, _HOP)
    shifted = jnp.concatenate([v3[:, 1:, :], vh], axis=1)
    o_ref[...] = jnp.concatenate(
        [v3 * w_ref[0, :], shifted * w_ref[1, :]], axis=2)


def _frames_group(x, w2, g, num_chunks, num_frames):
    nj = num_chunks // _KT

    full = pl.pallas_call(
        _frames_kernel,
        grid=(1, nj),
        in_specs=[
            pl.BlockSpec((8, _KT * _HOP), lambda i, j: (g, j)),
            pl.BlockSpec((8, _HOP),
                         lambda i, j: (g, jnp.minimum((j + 1) * _KT,
                                                      num_chunks - 1))),
            pl.BlockSpec((2, _HOP), lambda i, j: (0, 0)),
        ],
        out_specs=pl.BlockSpec((8, _KT, _SEG), lambda i, j: (0, j, 0)),
        out_shape=jax.ShapeDtypeStruct((8, num_chunks, _SEG), x.dtype),
    )(x, x, w2)
    return full[:, :num_frames, :]


def kernel(x, analysis_window):
    batch, num_samples = x.shape
    num_chunks = num_samples // _HOP
    num_frames = (num_samples - _SEG) // _HOP + 1
    w2 = analysis_window.reshape(2, _HOP)

    halves = [_frames_group(x, w2, g, num_chunks, num_frames)
              for g in range(batch // 8)]
    return jnp.concatenate(halves, axis=0)


# speedup vs baseline: 1.4457x; 1.4457x over previous
import jax
import jax.numpy as jnp
from jax.experimental import pallas as pl

_HOP = 256
_SEG = 512
_KT = 256


def _frames_kernel(x_ref, xh_ref, w_ref, o_ref):
    v3 = x_ref[...].reshape(8, _KT, _HOP)
    vh = xh_ref[...].reshape(8, 1, _HOP)
    shifted = jnp.concatenate([v3[:, 1:, :], vh], axis=1)
    o_ref[...] = jnp.concatenate(
        [v3 * w_ref[0, :], shifted * w_ref[1, :]], axis=2)


def kernel(x, analysis_window):
    batch, num_samples = x.shape
    num_chunks = num_samples // _HOP
    num_frames = (num_samples - _SEG) // _HOP + 1
    nj = num_chunks // _KT
    w2 = analysis_window.reshape(2, _HOP)

    full = pl.pallas_call(
        _frames_kernel,
        grid=(batch // 8, nj),
        in_specs=[
            pl.BlockSpec((8, _KT * _HOP), lambda i, j: (i, j)),
            pl.BlockSpec((8, _HOP),
                         lambda i, j: (i, jnp.minimum((j + 1) * _KT,
                                                      num_chunks - 1))),
            pl.BlockSpec((2, _HOP), lambda i, j: (0, 0)),
        ],
        out_specs=pl.BlockSpec((8, _KT, _SEG), lambda i, j: (i, j, 0)),
        out_shape=jax.ShapeDtypeStruct((batch, num_chunks, _SEG), x.dtype),
    )(x, x, w2)
    return full[:, :num_frames, :]


# speedup vs baseline: 1.4839x; 1.0264x over previous
import jax
import jax.numpy as jnp
from jax.experimental import pallas as pl

_HOP = 256
_SEG = 512
_KT = 512


def _frames_kernel(x_ref, xh_ref, w_ref, o_ref):
    v3 = x_ref[...].reshape(8, _KT, _HOP)
    vh = xh_ref[...].reshape(8, 1, _HOP)
    shifted = jnp.concatenate([v3[:, 1:, :], vh], axis=1)
    o_ref[...] = jnp.concatenate(
        [v3 * w_ref[0, :], shifted * w_ref[1, :]], axis=2)


def kernel(x, analysis_window):
    batch, num_samples = x.shape
    num_chunks = num_samples // _HOP
    num_frames = (num_samples - _SEG) // _HOP + 1
    nj = num_chunks // _KT
    w2 = analysis_window.reshape(2, _HOP)

    full = pl.pallas_call(
        _frames_kernel,
        grid=(batch // 8, nj),
        in_specs=[
            pl.BlockSpec((8, _KT * _HOP), lambda i, j: (i, j)),
            pl.BlockSpec((8, _HOP),
                         lambda i, j: (i, jnp.minimum((j + 1) * _KT,
                                                      num_chunks - 1))),
            pl.BlockSpec((2, _HOP), lambda i, j: (0, 0)),
        ],
        out_specs=pl.BlockSpec((8, _KT, _SEG), lambda i, j: (i, j, 0)),
        out_shape=jax.ShapeDtypeStruct((batch, num_chunks, _SEG), x.dtype),
    )(x, x, w2)
    return full[:, :num_frames, :]
